# baseline (speedup 1.0000x reference)
import functools

import jax
import jax.numpy as jnp
from jax import lax
from jax.experimental import pallas as pl
from jax.experimental.pallas import tpu as pltpu
from jax.experimental.pallas import tpu_sc as plsc

NUM_EMB = 1000000
DIM = 64
PDIM = 2 * DIM
B, F = 16384, 26
FLAT = B * F
NC, NS = 2, 16
NW = NC * NS
PER_W = FLAT // NW
CHUNK = 128
NCHUNK = PER_W // CHUNK
NBUF = 2

RB = 4000
SB = 64

_mesh = plsc.VectorSubcoreMesh(core_axis_name="c", subcore_axis_name="s")


def _pad_body(x_ref, o_ref):
    x = x_ref[...]
    o_ref[:, :DIM] = x
    o_ref[:, DIM:] = x


_pad_kernel = pl.pallas_call(
    _pad_body,
    grid=(NUM_EMB // RB,),
    in_specs=[pl.BlockSpec((RB, DIM), lambda i: (i, 0))],
    out_specs=pl.BlockSpec((RB, PDIM), lambda i: (i, 0)),
    out_shape=jax.ShapeDtypeStruct((NUM_EMB, PDIM), jnp.float32),
)


def _unpad_body(x_ref, o_ref):
    o_ref[...] = x_ref[:, :DIM].reshape(SB, F, DIM)


_unpad_kernel = pl.pallas_call(
    _unpad_body,
    grid=(B // SB,),
    in_specs=[pl.BlockSpec((SB * F, PDIM), lambda i: (i, 0))],
    out_specs=pl.BlockSpec((SB, F, DIM), lambda i: (i, 0, 0)),
    out_shape=jax.ShapeDtypeStruct((B, F, DIM), jnp.float32),
)


@functools.partial(
    pl.kernel,
    out_type=jax.ShapeDtypeStruct((FLAT, PDIM), jnp.float32),
    mesh=_mesh,
    scratch_types=[
        pltpu.VMEM((NCHUNK, CHUNK), jnp.int32),
        pltpu.VMEM((NBUF, CHUNK, PDIM), jnp.float32),
        pltpu.SemaphoreType.DMA((NBUF,)),
        pltpu.SemaphoreType.DMA((NBUF,)),
    ],
    compiler_params=pltpu.CompilerParams(use_tc_tiling_on_sc=True),
)
def _gather_kernel(idx_hbm, tabp_hbm, outp_hbm, idx_v, rows_v, gsem, ssem):
    wid = lax.axis_index("s") * NC + lax.axis_index("c")
    base = wid * PER_W

    pltpu.sync_copy(idx_hbm.at[wid], idx_v)

    def gather_start(chunk, buf):
        pltpu.async_copy(tabp_hbm.at[idx_v.at[chunk]], rows_v.at[buf],
                         gsem.at[buf])

    def gather_wait(chunk, buf):
        pltpu.make_async_copy(tabp_hbm.at[idx_v.at[chunk]], rows_v.at[buf],
                              gsem.at[buf]).wait()

    def store_start(chunk, buf):
        pltpu.async_copy(rows_v.at[buf],
                         outp_hbm.at[pl.ds(base + chunk * CHUNK, CHUNK)],
                         ssem.at[buf])

    def store_wait(chunk, buf):
        pltpu.make_async_copy(rows_v.at[buf],
                              outp_hbm.at[pl.ds(base + chunk * CHUNK, CHUNK)],
                              ssem.at[buf]).wait()

    for b in range(NBUF):
        gather_start(b, b)

    @pl.loop(0, NCHUNK, step=NBUF)
    def _grp(g):
        for b in range(NBUF):
            chunk = g + b
            gather_wait(chunk, b)
            store_start(chunk, b)
            nxt = chunk + NBUF

            @pl.when(nxt < NCHUNK)
            def _():
                store_wait(chunk, b)
                gather_start(nxt, b)

    for b in range(NBUF):
        store_wait(NCHUNK - NBUF + b, b)


def kernel(indices, table):
    idx3 = indices.reshape(NW, NCHUNK, CHUNK).astype(jnp.int32)
    tabp = _pad_kernel(table)
    outp = _gather_kernel(idx3, tabp)
    return _unpad_kernel(outp)


# speedup vs baseline: 1.4055x; 1.4055x over previous
import functools

import jax
import jax.numpy as jnp
from jax import lax
from jax.experimental import pallas as pl
from jax.experimental.pallas import tpu as pltpu
from jax.experimental.pallas import tpu_sc as plsc

NUM_EMB = 1000000
DIM = 64
B, F = 16384, 26
FLAT = B * F
NC, NS = 2, 16
NW = NC * NS
PER_W = FLAT // NW
CHUNK = 512
NCHUNK = PER_W // CHUNK
NBUF = 2

_mesh = plsc.VectorSubcoreMesh(core_axis_name="c", subcore_axis_name="s")


@functools.partial(
    pl.kernel,
    out_type=jax.ShapeDtypeStruct((FLAT, DIM), jnp.float32),
    mesh=_mesh,
    scratch_types=[
        pltpu.VMEM((NCHUNK, CHUNK), jnp.int32),
        pltpu.VMEM((NBUF, CHUNK, DIM), jnp.float32),
        pltpu.SemaphoreType.DMA((NBUF,)),
        pltpu.SemaphoreType.DMA((NBUF,)),
    ],
    compiler_params=pltpu.CompilerParams(use_tc_tiling_on_sc=False),
)
def _gather_kernel(idx_hbm, table_hbm, out_hbm, idx_v, rows_v, gsem, ssem):
    wid = lax.axis_index("s") * NC + lax.axis_index("c")
    base = wid * PER_W

    pltpu.sync_copy(idx_hbm.at[wid], idx_v)

    def gather_start(chunk, buf):
        pltpu.async_copy(table_hbm.at[idx_v.at[chunk]], rows_v.at[buf],
                         gsem.at[buf])

    def gather_wait(chunk, buf):
        pltpu.make_async_copy(table_hbm.at[idx_v.at[chunk]], rows_v.at[buf],
                              gsem.at[buf]).wait()

    def store_start(chunk, buf):
        pltpu.async_copy(rows_v.at[buf],
                         out_hbm.at[pl.ds(base + chunk * CHUNK, CHUNK)],
                         ssem.at[buf])

    def store_wait(chunk, buf):
        pltpu.make_async_copy(rows_v.at[buf],
                              out_hbm.at[pl.ds(base + chunk * CHUNK, CHUNK)],
                              ssem.at[buf]).wait()

    for b in range(NBUF):
        gather_start(b, b)

    @pl.loop(0, NCHUNK, step=NBUF)
    def _grp(g):
        for b in range(NBUF):
            chunk = g + b
            gather_wait(chunk, b)
            store_start(chunk, b)
            nxt = chunk + NBUF

            @pl.when(nxt < NCHUNK)
            def _():
                store_wait(chunk, b)
                gather_start(nxt, b)

    for b in range(NBUF):
        store_wait(NCHUNK - NBUF + b, b)


def kernel(indices, table):
    flat = indices.reshape(NW, NCHUNK, CHUNK).astype(jnp.int32)
    out = _gather_kernel(flat, table)
    return out.reshape(B, F, DIM)


# speedup vs baseline: 1.4071x; 1.0012x over previous
import functools

import jax
import jax.numpy as jnp
from jax import lax
from jax.experimental import pallas as pl
from jax.experimental.pallas import tpu as pltpu
from jax.experimental.pallas import tpu_sc as plsc

NUM_EMB = 1000000
DIM = 64
B, F = 16384, 26
FLAT = B * F
NC, NS = 2, 16
NW = NC * NS
PER_W = FLAT // NW
CHUNK = 512
NCHUNK = PER_W // CHUNK
NBUF = 2

_mesh = plsc.VectorSubcoreMesh(core_axis_name="c", subcore_axis_name="s")


@functools.partial(
    pl.kernel,
    out_type=jax.ShapeDtypeStruct((FLAT, DIM), jnp.float32),
    mesh=_mesh,
    scratch_types=[
        pltpu.VMEM((NCHUNK, CHUNK), jnp.int32),
        pltpu.VMEM((NBUF, CHUNK, DIM), jnp.float32),
        pltpu.SemaphoreType.DMA((NBUF,)),
        pltpu.SemaphoreType.DMA((NBUF,)),
    ],
    compiler_params=pltpu.CompilerParams(use_tc_tiling_on_sc=False,
                                         allow_input_fusion=[False, True]),
)
def _gather_kernel(idx_hbm, table_hbm, out_hbm, idx_v, rows_v, gsem, ssem):
    wid = lax.axis_index("s") * NC + lax.axis_index("c")
    base = wid * PER_W

    pltpu.sync_copy(idx_hbm.at[wid], idx_v)

    def gather_start(chunk, buf):
        pltpu.async_copy(table_hbm.at[idx_v.at[chunk]], rows_v.at[buf],
                         gsem.at[buf])

    def gather_wait(chunk, buf):
        pltpu.make_async_copy(table_hbm.at[idx_v.at[chunk]], rows_v.at[buf],
                              gsem.at[buf]).wait()

    def store_start(chunk, buf):
        pltpu.async_copy(rows_v.at[buf],
                         out_hbm.at[pl.ds(base + chunk * CHUNK, CHUNK)],
                         ssem.at[buf])

    def store_wait(chunk, buf):
        pltpu.make_async_copy(rows_v.at[buf],
                              out_hbm.at[pl.ds(base + chunk * CHUNK, CHUNK)],
                              ssem.at[buf]).wait()

    for b in range(NBUF):
        gather_start(b, b)

    @pl.loop(0, NCHUNK, step=NBUF)
    def _grp(g):
        for b in range(NBUF):
            chunk = g + b
            gather_wait(chunk, b)
            store_start(chunk, b)
            nxt = chunk + NBUF

            @pl.when(nxt < NCHUNK)
            def _():
                store_wait(chunk, b)
                gather_start(nxt, b)

    for b in range(NBUF):
        store_wait(NCHUNK - NBUF + b, b)


def kernel(indices, table):
    flat = indices.reshape(NW, NCHUNK, CHUNK).astype(jnp.int32)
    out = _gather_kernel(flat, table)
    return out.reshape(B, F, DIM)
